# baseline (speedup 1.0000x reference)
import functools

import jax
import jax.numpy as jnp
from jax import lax
from jax.experimental import pallas as pl
from jax.experimental.pallas import tpu as pltpu
from jax.experimental.pallas import tpu_sc as plsc

N, E, D, H = 10000, 320000, 128, 32
NW = 32
EW = E // NW
K = 80
ITERS = EW // K
RPS = N // 16
ZR = 25
BN = 1000


def _sc_segsum(table, src3, dst3, width, nbuf):
    mesh = plsc.VectorSubcoreMesh(core_axis_name="c", subcore_axis_name="s")

    @functools.partial(
        pl.kernel,
        mesh=mesh,
        out_type=jax.ShapeDtypeStruct((2, N, width), jnp.float32),
        compiler_params=pltpu.CompilerParams(use_tc_tiling_on_sc=False),
        scratch_types=(
            [pltpu.VMEM((ITERS, K), jnp.int32),
             pltpu.VMEM((ITERS, K), jnp.int32)]
            + [pltpu.VMEM((K, width), jnp.float32) for _ in range(nbuf)]
            + [pltpu.VMEM_SHARED((N, width), jnp.float32)]
            + [pltpu.SemaphoreType.DMA for _ in range(nbuf)]
        ),
    )
    def k(tab_hbm, src_hbm, dst_hbm, out_hbm, sidx, didx, *rest):
        bufs = rest[:nbuf]
        acc = rest[nbuf]
        sems = rest[nbuf + 1:]
        cid = lax.axis_index("c")
        sid = lax.axis_index("s")
        wid = sid * 2 + cid

        pltpu.async_copy(src_hbm.at[wid], sidx, sems[0])
        pltpu.async_copy(dst_hbm.at[wid], didx, sems[1])

        z = jnp.zeros((16,), jnp.float32)

        def zrow(i, carry):
            for c in range(width // 16):
                bufs[0][i, pl.ds(c * 16, 16)] = z
            return carry

        lax.fori_loop(0, K, zrow, 0)

        for c in range(RPS // K):
            pltpu.sync_copy(bufs[0], acc.at[pl.ds(sid * RPS + c * K, K)])
        rem = RPS % K
        if rem:
            pltpu.sync_copy(bufs[0].at[pl.ds(0, rem)],
                            acc.at[pl.ds(sid * RPS + (RPS // K) * K, rem)])
        pltpu.make_async_copy(src_hbm.at[wid], sidx, sems[0]).wait()
        pltpu.make_async_copy(dst_hbm.at[wid], didx, sems[1]).wait()
        plsc.subcore_barrier()

        def gather(j, b):
            pltpu.async_copy(tab_hbm.at[sidx.at[j]], bufs[b], sems[b])

        def drain(b):
            pltpu.make_async_copy(tab_hbm.at[sidx.at[0]], bufs[b], sems[b]).wait()

        def scat(j, b):
            pltpu.sync_copy(bufs[b], acc.at[didx.at[j]], add=True)

        for b in range(nbuf - 1):
            gather(b, b)

        main = (ITERS - 1) // nbuf

        def body(p, carry):
            for b in range(nbuf):
                j = nbuf * p + b
                nxt = j + nbuf - 1

                @pl.when(nxt < ITERS)
                def _():
                    gather(nxt, (b + nbuf - 1) % nbuf)

                drain(b)
                scat(j, b)
            return carry

        lax.fori_loop(0, main, body, 0)
        for t in range(nbuf * main, ITERS):
            drain(t % nbuf)
            scat(t, t % nbuf)
        plsc.subcore_barrier()

        pltpu.sync_copy(acc.at[pl.ds(sid * RPS, RPS)],
                        out_hbm.at[cid, pl.ds(sid * RPS, RPS)])

    return k(table, src3, dst3)


def _root_body(v_ref, w_ref, out_ref):
    out_ref[...] = jnp.dot(v_ref[...], w_ref[...],
                           preferred_element_type=jnp.float32)


def _tc_root(v, w):
    din, dout = w.shape
    return pl.pallas_call(
        _root_body,
        grid=(N // BN,),
        in_specs=[
            pl.BlockSpec((BN, din), lambda i: (i, 0)),
            pl.BlockSpec((din, dout), lambda i: (0, 0)),
        ],
        out_specs=pl.BlockSpec((BN, dout), lambda i: (i, 0)),
        out_shape=jax.ShapeDtypeStruct((N, dout), jnp.float32),
    )(v, w)


def _combine_body(p_ref, wrel_ref, b_ref, root_ref, out_ref):
    agg = p_ref[0] + p_ref[1]
    out_ref[...] = jnp.maximum(
        jnp.dot(agg, wrel_ref[...], preferred_element_type=jnp.float32)
        + b_ref[...]
        + root_ref[...],
        0.0,
    )


def _tc_combine(parts, W_rel, b, root):
    din, dout = W_rel.shape
    return pl.pallas_call(
        _combine_body,
        grid=(N // BN,),
        in_specs=[
            pl.BlockSpec((2, BN, din), lambda i: (0, i, 0)),
            pl.BlockSpec((din, dout), lambda i: (0, 0)),
            pl.BlockSpec((1, dout), lambda i: (0, 0)),
            pl.BlockSpec((BN, dout), lambda i: (i, 0)),
        ],
        out_specs=pl.BlockSpec((BN, dout), lambda i: (i, 0)),
        out_shape=jax.ShapeDtypeStruct((N, dout), jnp.float32),
    )(parts, W_rel, b.reshape(1, dout), root)


def kernel(x, edge_index, W1_rel, b1, W1_root, W2_rel, b2, W2_root):
    src3 = edge_index[0].reshape(NW, ITERS, K)
    dst3 = edge_index[1].reshape(NW, ITERS, K)
    xroot = _tc_root(x, W1_root)
    parts1 = _sc_segsum(x, src3, dst3, D, nbuf=3)
    h = _tc_combine(parts1, W1_rel, b1, xroot)
    hroot = _tc_root(h, W2_root)
    parts2 = _sc_segsum(h, src3, dst3, H, nbuf=8)
    out = _tc_combine(parts2, W2_rel, b2, hroot)
    return out


# speedup vs baseline: 1.0555x; 1.0555x over previous
import functools

import jax
import jax.numpy as jnp
from jax import lax
from jax.experimental import pallas as pl
from jax.experimental.pallas import tpu as pltpu
from jax.experimental.pallas import tpu_sc as plsc

N, E, D, H = 10000, 320000, 128, 32
NW = 32
EW = E // NW
K = 80
ITERS = EW // K
RPS = N // 16
ZR = 25
BN = 1000


def _sc_segsum(table, src3, dst3, width, nbuf, tc_tiling=False):
    mesh = plsc.VectorSubcoreMesh(core_axis_name="c", subcore_axis_name="s")

    @functools.partial(
        pl.kernel,
        mesh=mesh,
        out_type=jax.ShapeDtypeStruct((2, N, width), jnp.float32),
        compiler_params=pltpu.CompilerParams(use_tc_tiling_on_sc=tc_tiling),
        scratch_types=(
            [pltpu.VMEM((ITERS, K), jnp.int32),
             pltpu.VMEM((ITERS, K), jnp.int32)]
            + [pltpu.VMEM((K, width), jnp.float32) for _ in range(nbuf)]
            + [pltpu.VMEM_SHARED((N, width), jnp.float32)]
            + [pltpu.SemaphoreType.DMA for _ in range(nbuf)]
        ),
    )
    def k(tab_hbm, src_hbm, dst_hbm, out_hbm, sidx, didx, *rest):
        bufs = rest[:nbuf]
        acc = rest[nbuf]
        sems = rest[nbuf + 1:]
        cid = lax.axis_index("c")
        sid = lax.axis_index("s")
        wid = sid * 2 + cid

        pltpu.async_copy(src_hbm.at[wid], sidx, sems[0])
        pltpu.async_copy(dst_hbm.at[wid], didx, sems[1])

        z = jnp.zeros((16,), jnp.float32)

        def zrow(i, carry):
            for c in range(width // 16):
                bufs[0][i, pl.ds(c * 16, 16)] = z
            return carry

        lax.fori_loop(0, K, zrow, 0)

        for c in range(RPS // K):
            pltpu.sync_copy(bufs[0], acc.at[pl.ds(sid * RPS + c * K, K)])
        rem = RPS % K
        if rem:
            pltpu.sync_copy(bufs[0].at[pl.ds(0, rem)],
                            acc.at[pl.ds(sid * RPS + (RPS // K) * K, rem)])
        pltpu.make_async_copy(src_hbm.at[wid], sidx, sems[0]).wait()
        pltpu.make_async_copy(dst_hbm.at[wid], didx, sems[1]).wait()
        plsc.subcore_barrier()

        def gather(j, b):
            pltpu.async_copy(tab_hbm.at[sidx.at[j]], bufs[b], sems[b])

        def drain(b):
            pltpu.make_async_copy(tab_hbm.at[sidx.at[0]], bufs[b], sems[b]).wait()

        def scat(j, b):
            pltpu.sync_copy(bufs[b], acc.at[didx.at[j]], add=True)

        for b in range(nbuf - 1):
            gather(b, b)

        main = (ITERS - 1) // nbuf

        def body(p, carry):
            for b in range(nbuf):
                j = nbuf * p + b
                nxt = j + nbuf - 1

                @pl.when(nxt < ITERS)
                def _():
                    gather(nxt, (b + nbuf - 1) % nbuf)

                drain(b)
                scat(j, b)
            return carry

        lax.fori_loop(0, main, body, 0)
        for t in range(nbuf * main, ITERS):
            drain(t % nbuf)
            scat(t, t % nbuf)
        plsc.subcore_barrier()

        pltpu.sync_copy(acc.at[pl.ds(sid * RPS, RPS)],
                        out_hbm.at[cid, pl.ds(sid * RPS, RPS)])

    return k(table, src3, dst3)


def _layer_body(p_ref, v_ref, wrel_ref, wroot_ref, b_ref, out_ref):
    agg = p_ref[0] + p_ref[1]
    out_ref[...] = jnp.maximum(
        jnp.dot(agg, wrel_ref[...], preferred_element_type=jnp.float32)
        + b_ref[...]
        + jnp.dot(v_ref[...], wroot_ref[...], preferred_element_type=jnp.float32),
        0.0,
    )


def _tc_layer(parts, v, W_rel, W_root, b):
    din, dout = W_rel.shape
    return pl.pallas_call(
        _layer_body,
        grid=(N // BN,),
        in_specs=[
            pl.BlockSpec((2, BN, din), lambda i: (0, i, 0)),
            pl.BlockSpec((BN, din), lambda i: (i, 0)),
            pl.BlockSpec((din, dout), lambda i: (0, 0)),
            pl.BlockSpec((din, dout), lambda i: (0, 0)),
            pl.BlockSpec((1, dout), lambda i: (0, 0)),
        ],
        out_specs=pl.BlockSpec((BN, dout), lambda i: (i, 0)),
        out_shape=jax.ShapeDtypeStruct((N, dout), jnp.float32),
    )(parts, v, W_rel, W_root, b.reshape(1, dout))


def kernel(x, edge_index, W1_rel, b1, W1_root, W2_rel, b2, W2_root):
    src3 = edge_index[0].reshape(NW, ITERS, K)
    dst3 = edge_index[1].reshape(NW, ITERS, K)
    parts1 = _sc_segsum(x, src3, dst3, D, nbuf=3)
    h = _tc_layer(parts1, x, W1_rel, W1_root, b1)
    parts2 = _sc_segsum(h, src3, dst3, H, nbuf=8)
    out = _tc_layer(parts2, h, W2_rel, W2_root, b2)
    return out
